# baseline (speedup 1.0000x reference)
import functools

import jax
import jax.numpy as jnp
from jax import lax
from jax.experimental import pallas as pl
from jax.experimental.pallas import tpu as pltpu
from jax.experimental.pallas import tpu_sc as plsc

_LANES = 16
_OUT_CHUNK = 2048


@functools.lru_cache(maxsize=None)
def _make_gather(batch: int, embed_dim: int, num_tasks: int):
    info = plsc.get_sparse_core_info()
    nc, ns = info.num_cores, info.num_subcores
    nw = nc * ns
    dims_per_w = embed_dim // nw
    n_chunk = batch // _OUT_CHUNK
    unroll = 16
    mesh = plsc.VectorSubcoreMesh(core_axis_name="c", subcore_axis_name="s")

    @functools.partial(
        pl.kernel,
        mesh=mesh,
        out_type=jax.ShapeDtypeStruct((embed_dim, batch), jnp.float32),
        scratch_types=[
            pltpu.VMEM((batch,), jnp.int32),
            pltpu.VMEM((num_tasks,), jnp.float32),
            pltpu.VMEM((_OUT_CHUNK,), jnp.float32),
            pltpu.VMEM((_OUT_CHUNK,), jnp.float32),
            pltpu.SemaphoreType.DMA,
            pltpu.SemaphoreType.DMA,
        ],
        compiler_params=pltpu.CompilerParams(needs_layout_passes=False),
    )
    def gather_kernel(idx_hbm, tablet_hbm, outt_hbm, ids_v, row_v,
                      out_a, out_b, out_sem, row_sem):
        wid = lax.axis_index("s") * nc + lax.axis_index("c")
        d0 = wid * dims_per_w
        pltpu.make_async_copy(tablet_hbm.at[d0], row_v, row_sem).start()
        pltpu.sync_copy(idx_hbm, ids_v)
        n_inflight = 0
        for j in range(dims_per_w):
            d = d0 + j
            pltpu.make_async_copy(tablet_hbm.at[d], row_v, row_sem).wait()
            for c in range(n_chunk):
                out_v = out_a if (j * n_chunk + c) % 2 == 0 else out_b

                def body(o, c=c, out_v=out_v):
                    idx16 = ids_v[pl.ds(c * _OUT_CHUNK + o, _LANES)]
                    out_v[pl.ds(o, _LANES)] = plsc.load_gather(
                        row_v, [idx16]
                    )

                if n_inflight >= 2:
                    pltpu.make_async_copy(
                        outt_hbm.at[0, pl.ds(0, _OUT_CHUNK)],
                        out_v,
                        out_sem,
                    ).wait()
                    n_inflight -= 1
                plsc.parallel_loop(
                    0, _OUT_CHUNK, step=_LANES, unroll=unroll
                )(body)
                if c == n_chunk - 1 and j + 1 < dims_per_w:
                    pltpu.make_async_copy(
                        tablet_hbm.at[d + 1], row_v, row_sem
                    ).start()
                pltpu.make_async_copy(
                    out_v,
                    outt_hbm.at[d, pl.ds(c * _OUT_CHUNK, _OUT_CHUNK)],
                    out_sem,
                ).start()
                n_inflight += 1
        for _ in range(n_inflight):
            pltpu.make_async_copy(
                outt_hbm.at[0, pl.ds(0, _OUT_CHUNK)],
                out_a,
                out_sem,
            ).wait()

    return gather_kernel


def kernel(task_ids, embedding_weight):
    if task_ids.ndim == 2:
        task_ids = task_ids[:, 0]
    batch = task_ids.shape[0]
    num_tasks, embed_dim = embedding_weight.shape
    fn = _make_gather(batch, embed_dim, num_tasks)
    outt = fn(task_ids.astype(jnp.int32), embedding_weight.T)
    return outt.T


# speedup vs baseline: 1.0285x; 1.0285x over previous
import functools

import jax
import jax.numpy as jnp
from jax import lax
from jax.experimental import pallas as pl
from jax.experimental.pallas import tpu as pltpu
from jax.experimental.pallas import tpu_sc as plsc

_LANES = 16
_OUT_CHUNK = 2048


@functools.lru_cache(maxsize=None)
def _make_gather(batch: int, embed_dim: int, num_tasks: int):
    info = plsc.get_sparse_core_info()
    nc, ns = info.num_cores, info.num_subcores
    nw = nc * ns
    dims_per_w = embed_dim // nw
    n_chunk = batch // _OUT_CHUNK
    unroll = 4
    mesh = plsc.VectorSubcoreMesh(core_axis_name="c", subcore_axis_name="s")

    @functools.partial(
        pl.kernel,
        mesh=mesh,
        out_type=jax.ShapeDtypeStruct((embed_dim, batch), jnp.float32),
        scratch_types=[
            pltpu.VMEM((batch,), jnp.int32),
            pltpu.VMEM((num_tasks,), jnp.float32),
            pltpu.VMEM((_OUT_CHUNK,), jnp.float32),
            pltpu.VMEM((_OUT_CHUNK,), jnp.float32),
            pltpu.SemaphoreType.DMA,
            pltpu.SemaphoreType.DMA,
        ],
        compiler_params=pltpu.CompilerParams(needs_layout_passes=False),
    )
    def gather_kernel(idx_hbm, tablet_hbm, outt_hbm, ids_v, row_v,
                      out_a, out_b, out_sem, row_sem):
        wid = lax.axis_index("s") * nc + lax.axis_index("c")
        d0 = wid * dims_per_w
        pltpu.make_async_copy(tablet_hbm.at[d0], row_v, row_sem).start()
        pltpu.sync_copy(idx_hbm, ids_v)
        n_inflight = 0
        for j in range(dims_per_w):
            d = d0 + j
            pltpu.make_async_copy(tablet_hbm.at[d], row_v, row_sem).wait()
            for c in range(n_chunk):
                out_v = out_a if (j * n_chunk + c) % 2 == 0 else out_b

                def body(o, c=c, out_v=out_v):
                    idx16 = ids_v[pl.ds(c * _OUT_CHUNK + o, _LANES)]
                    out_v[pl.ds(o, _LANES)] = plsc.load_gather(
                        row_v, [idx16]
                    )

                if n_inflight >= 2:
                    pltpu.make_async_copy(
                        outt_hbm.at[0, pl.ds(0, _OUT_CHUNK)],
                        out_v,
                        out_sem,
                    ).wait()
                    n_inflight -= 1
                plsc.parallel_loop(
                    0, _OUT_CHUNK, step=_LANES, unroll=unroll
                )(body)
                if c == n_chunk - 1 and j + 1 < dims_per_w:
                    pltpu.make_async_copy(
                        tablet_hbm.at[d + 1], row_v, row_sem
                    ).start()
                pltpu.make_async_copy(
                    out_v,
                    outt_hbm.at[d, pl.ds(c * _OUT_CHUNK, _OUT_CHUNK)],
                    out_sem,
                ).start()
                n_inflight += 1
        for _ in range(n_inflight):
            pltpu.make_async_copy(
                outt_hbm.at[0, pl.ds(0, _OUT_CHUNK)],
                out_a,
                out_sem,
            ).wait()

    return gather_kernel


def kernel(task_ids, embedding_weight):
    if task_ids.ndim == 2:
        task_ids = task_ids[:, 0]
    batch = task_ids.shape[0]
    num_tasks, embed_dim = embedding_weight.shape
    fn = _make_gather(batch, embed_dim, num_tasks)
    outt = fn(task_ids.astype(jnp.int32), embedding_weight.T)
    return outt.T


# speedup vs baseline: 1.0443x; 1.0153x over previous
import functools

import jax
import jax.numpy as jnp
from jax import lax
from jax.experimental import pallas as pl
from jax.experimental.pallas import tpu as pltpu
from jax.experimental.pallas import tpu_sc as plsc

_LANES = 16
_OUT_CHUNK = 4096


@functools.lru_cache(maxsize=None)
def _make_gather(batch: int, embed_dim: int, num_tasks: int):
    info = plsc.get_sparse_core_info()
    nc, ns = info.num_cores, info.num_subcores
    nw = nc * ns
    dims_per_w = embed_dim // nw
    n_chunk = batch // _OUT_CHUNK
    unroll = 4
    mesh = plsc.VectorSubcoreMesh(core_axis_name="c", subcore_axis_name="s")

    @functools.partial(
        pl.kernel,
        mesh=mesh,
        out_type=jax.ShapeDtypeStruct((embed_dim, batch), jnp.float32),
        scratch_types=[
            pltpu.VMEM((batch,), jnp.int32),
            pltpu.VMEM((num_tasks,), jnp.float32),
            pltpu.VMEM((_OUT_CHUNK,), jnp.float32),
            pltpu.VMEM((_OUT_CHUNK,), jnp.float32),
            pltpu.SemaphoreType.DMA,
            pltpu.SemaphoreType.DMA,
        ],
        compiler_params=pltpu.CompilerParams(needs_layout_passes=False),
    )
    def gather_kernel(idx_hbm, tablet_hbm, outt_hbm, ids_v, row_v,
                      out_a, out_b, out_sem, row_sem):
        wid = lax.axis_index("s") * nc + lax.axis_index("c")
        d0 = wid * dims_per_w
        pltpu.make_async_copy(tablet_hbm.at[d0], row_v, row_sem).start()
        pltpu.sync_copy(idx_hbm, ids_v)
        n_inflight = 0
        for j in range(dims_per_w):
            d = d0 + j
            pltpu.make_async_copy(tablet_hbm.at[d], row_v, row_sem).wait()
            for c in range(n_chunk):
                out_v = out_a if (j * n_chunk + c) % 2 == 0 else out_b

                def body(o, c=c, out_v=out_v):
                    idx16 = ids_v[pl.ds(c * _OUT_CHUNK + o, _LANES)]
                    out_v[pl.ds(o, _LANES)] = plsc.load_gather(
                        row_v, [idx16]
                    )

                if n_inflight >= 2:
                    pltpu.make_async_copy(
                        outt_hbm.at[0, pl.ds(0, _OUT_CHUNK)],
                        out_v,
                        out_sem,
                    ).wait()
                    n_inflight -= 1
                plsc.parallel_loop(
                    0, _OUT_CHUNK, step=_LANES, unroll=unroll
                )(body)
                if c == n_chunk - 1 and j + 1 < dims_per_w:
                    pltpu.make_async_copy(
                        tablet_hbm.at[d + 1], row_v, row_sem
                    ).start()
                pltpu.make_async_copy(
                    out_v,
                    outt_hbm.at[d, pl.ds(c * _OUT_CHUNK, _OUT_CHUNK)],
                    out_sem,
                ).start()
                n_inflight += 1
        for _ in range(n_inflight):
            pltpu.make_async_copy(
                outt_hbm.at[0, pl.ds(0, _OUT_CHUNK)],
                out_a,
                out_sem,
            ).wait()

    return gather_kernel


def kernel(task_ids, embedding_weight):
    if task_ids.ndim == 2:
        task_ids = task_ids[:, 0]
    batch = task_ids.shape[0]
    num_tasks, embed_dim = embedding_weight.shape
    fn = _make_gather(batch, embed_dim, num_tasks)
    outt = fn(task_ids.astype(jnp.int32), embedding_weight.T)
    return outt.T
